# Initial kernel scaffold; baseline (speedup 1.0000x reference)
import jax
import jax.numpy as jnp
from jax.experimental import pallas as pl


def kernel(inputs, emb_weight):
    raise NotImplementedError("write your pallas kernel here")


# speedup vs baseline: 6.2743x; 6.2743x over previous
import functools

import jax
import jax.numpy as jnp
from jax import lax
from jax.experimental import pallas as pl
from jax.experimental.pallas import tpu as pltpu
from jax.experimental.pallas import tpu_sc as plsc

MAXV = 11
D = 128
NLANE = 16
NVEC = D // NLANE


@functools.cache
def _build_sc_call(R: int):
    NC, NS = 2, 16
    NW = NC * NS
    RW = R // NW
    CH = 512
    NCH = RW // CH
    assert RW % CH == 0

    mesh = plsc.VectorSubcoreMesh(
        core_axis_name="c", subcore_axis_name="s", num_cores=NC, num_subcores=NS
    )

    @functools.partial(
        pl.kernel,
        out_type=jax.ShapeDtypeStruct((R, D), jnp.float32),
        mesh=mesh,
        scratch_types=[
            pltpu.VMEM((MAXV * 3, D), jnp.float32),
            pltpu.VMEM((MAXV * MAXV, D), jnp.float32),
            pltpu.VMEM((3, CH), jnp.int32),
            pltpu.VMEM((CH, D), jnp.float32),
        ],
    )
    def sc_embed(idx_hbm, tab_hbm, out_hbm, tab_v, t01_v, idx_v, out_v):
        wid = lax.axis_index("s") * NC + lax.axis_index("c")
        base = wid * RW

        pltpu.sync_copy(tab_hbm, tab_v)

        def build_a(a, _):
            def build_b(b, _):
                j = a * MAXV + b
                for k in range(NVEC):
                    s = pl.ds(k * NLANE, NLANE)
                    t01_v[j, s] = tab_v[a, s] + tab_v[MAXV + b, s]
                return 0

            lax.fori_loop(0, MAXV, build_b, 0)
            return 0

        lax.fori_loop(0, MAXV, build_a, 0)

        def chunk(g, _):
            cbase = base + g * CH
            pltpu.sync_copy(idx_hbm.at[:, pl.ds(cbase, CH)], idx_v)

            def group(gi, _):
                gb = gi * NLANE
                iv0 = idx_v[0, pl.ds(gb, NLANE)]
                iv1 = idx_v[1, pl.ds(gb, NLANE)]
                iv2 = idx_v[2, pl.ds(gb, NLANE)]
                a01v = iv0 * MAXV + iv1
                a2v = iv2 + 2 * MAXV
                for j in range(NLANE):
                    a01 = a01v[j]
                    a2 = a2v[j]
                    for k in range(NVEC):
                        s = pl.ds(k * NLANE, NLANE)
                        out_v[gb + j, s] = t01_v[a01, s] + tab_v[a2, s]
                return 0

            lax.fori_loop(0, CH // NLANE, group, 0)
            pltpu.sync_copy(out_v, out_hbm.at[pl.ds(cbase, CH), :])
            return 0

        lax.fori_loop(0, NCH, chunk, 0)

    return sc_embed


def kernel(inputs, emb_weight):
    B, X, Y, C = inputs.shape
    R = B * X * Y
    idx = inputs.reshape(R, C).astype(jnp.int32).T
    out = _build_sc_call(R)(idx, emb_weight)
    return out.reshape(B, X, Y, D)


# speedup vs baseline: 12.6021x; 2.0085x over previous
import functools

import jax
import jax.numpy as jnp
from jax import lax
from jax.experimental import pallas as pl
from jax.experimental.pallas import tpu as pltpu
from jax.experimental.pallas import tpu_sc as plsc

MAXV = 11
D = 128
NLANE = 16
NVEC = D // NLANE
NCOMBO = MAXV * MAXV * MAXV
BLD = 84
GSZ = 128


@functools.cache
def _build_sc_call(R: int):
    NC, NS = 2, 16
    NW = NC * NS
    RW = R // NW
    CH = 256
    NCH = RW // CH
    assert RW % CH == 0 and NCH % 2 == 0 and CH == 2 * GSZ

    mesh = plsc.VectorSubcoreMesh(
        core_axis_name="c", subcore_axis_name="s", num_cores=NC, num_subcores=NS
    )

    @functools.partial(
        pl.kernel,
        out_type=jax.ShapeDtypeStruct((R, D), jnp.float32),
        mesh=mesh,
        scratch_types=[
            pltpu.VMEM((3 * MAXV, D), jnp.float32),
            pltpu.VMEM((BLD, D), jnp.float32),
            pltpu.VMEM_SHARED((16 * BLD, D), jnp.float32),
            pltpu.VMEM((2, 3, CH), jnp.int32),
            pltpu.VMEM((2, 2, GSZ), jnp.int32),
            pltpu.VMEM((CH, D), jnp.float32),
            pltpu.VMEM((CH, D), jnp.float32),
            pltpu.SemaphoreType.DMA,
            pltpu.SemaphoreType.DMA,
            pltpu.SemaphoreType.DMA,
            pltpu.SemaphoreType.DMA,
            pltpu.SemaphoreType.DMA,
            pltpu.SemaphoreType.DMA,
        ],
    )
    def sc_embed(idx_hbm, tab_hbm, out_hbm, tab_v, bld_v, t012_sh, idx_v,
                 cidx_v, out_v0, out_v1, in_s0, in_s1, g_s0, g_s1, o_s0, o_s1):
        cid = lax.axis_index("c")
        sid = lax.axis_index("s")
        wid = sid * NC + cid
        base = wid * RW
        out_v = (out_v0, out_v1)
        in_s = (in_s0, in_s1)
        g_s = (g_s0, g_s1)
        o_s = (o_s0, o_s1)

        pltpu.sync_copy(tab_hbm, tab_v)

        r0 = sid * BLD
        a0 = r0 // (MAXV * MAXV)
        rem = r0 - a0 * (MAXV * MAXV)
        b0 = rem // MAXV
        c0 = rem - b0 * MAXV

        def build(j, abc):
            a, b, c = abc
            for k in range(NVEC):
                s = pl.ds(k * NLANE, NLANE)
                bld_v[j, s] = tab_v[a, s] + tab_v[MAXV + b, s] + tab_v[2 * MAXV + c, s]
            roll_c = c == MAXV - 1
            roll_b = jnp.logical_and(roll_c, b == MAXV - 1)
            c = jnp.where(roll_c, 0, c + 1)
            b = jnp.where(roll_c, jnp.where(roll_b, 0, b + 1), b)
            a = jnp.where(roll_b, a + 1, a)
            return (a, b, c)

        lax.fori_loop(0, BLD, build, (a0, b0, c0))
        pltpu.sync_copy(bld_v, t012_sh.at[pl.ds(r0, BLD), :])
        plsc.subcore_barrier()

        def start_in(g, b):
            return pltpu.async_copy(
                idx_hbm.at[:, pl.ds(base + g * CH, CH)], idx_v.at[b], in_s[b]
            )

        start_in(0, 0)

        def step(t, _):
            for b in range(2):
                g = 2 * t + b
                pltpu.make_async_copy(
                    idx_hbm.at[:, pl.ds(base, CH)], idx_v.at[b], in_s[b]
                ).wait()

                @pl.when(g + 1 < NCH)
                def _():
                    start_in(g + 1, 1 - b)

                for gi in range(CH // NLANE):
                    q, off = divmod(gi * NLANE, GSZ)
                    iv0 = idx_v[b, 0, pl.ds(gi * NLANE, NLANE)]
                    iv1 = idx_v[b, 1, pl.ds(gi * NLANE, NLANE)]
                    iv2 = idx_v[b, 2, pl.ds(gi * NLANE, NLANE)]
                    cidx_v[b, q, pl.ds(off, NLANE)] = (
                        iv0 * MAXV + iv1
                    ) * MAXV + iv2

                @pl.when(g >= 2)
                def _():
                    pltpu.make_async_copy(
                        out_v[b], out_hbm.at[pl.ds(base, CH), :], o_s[b]
                    ).wait()

                for q in range(2):
                    pltpu.async_copy(
                        t012_sh.at[cidx_v.at[b, q]],
                        out_v[b].at[pl.ds(q * GSZ, GSZ), :],
                        g_s[b],
                    ).wait()
                pltpu.async_copy(
                    out_v[b], out_hbm.at[pl.ds(base + g * CH, CH), :], o_s[b]
                )
            return 0

        lax.fori_loop(0, NCH // 2, step, 0)
        for b in range(2):
            pltpu.make_async_copy(
                out_v[b], out_hbm.at[pl.ds(base, CH), :], o_s[b]
            ).wait()

    return sc_embed


def kernel(inputs, emb_weight):
    B, X, Y, C = inputs.shape
    R = B * X * Y
    idx = inputs.reshape(R, C).astype(jnp.int32).T
    out = _build_sc_call(R)(idx, emb_weight)
    return out.reshape(B, X, Y, D)


# speedup vs baseline: 12.6097x; 1.0006x over previous
import functools

import jax
import jax.numpy as jnp
from jax import lax
from jax.experimental import pallas as pl
from jax.experimental.pallas import tpu as pltpu
from jax.experimental.pallas import tpu_sc as plsc

MAXV = 11
D = 128
NLANE = 16
NVEC = D // NLANE
NCOMBO = MAXV * MAXV * MAXV
BLD = 84
GSZ = 128


@functools.cache
def _build_sc_call(R: int):
    NC, NS = 2, 16
    NW = NC * NS
    RW = R // NW
    CH = 256
    NCH = RW // CH
    assert RW % CH == 0 and NCH % 2 == 0 and CH == 2 * GSZ

    mesh = plsc.VectorSubcoreMesh(
        core_axis_name="c", subcore_axis_name="s", num_cores=NC, num_subcores=NS
    )

    @functools.partial(
        pl.kernel,
        out_type=jax.ShapeDtypeStruct((R, D), jnp.float32),
        mesh=mesh,
        compiler_params=pltpu.CompilerParams(use_tc_tiling_on_sc=True),
        scratch_types=[
            pltpu.VMEM((3 * MAXV, D), jnp.float32),
            pltpu.VMEM((BLD, D), jnp.float32),
            pltpu.VMEM_SHARED((16 * BLD, D), jnp.float32),
            pltpu.VMEM((2, 3, CH), jnp.int32),
            pltpu.VMEM((2, 2, GSZ), jnp.int32),
            pltpu.VMEM((CH, D), jnp.float32),
            pltpu.VMEM((CH, D), jnp.float32),
            pltpu.SemaphoreType.DMA,
            pltpu.SemaphoreType.DMA,
            pltpu.SemaphoreType.DMA,
            pltpu.SemaphoreType.DMA,
            pltpu.SemaphoreType.DMA,
            pltpu.SemaphoreType.DMA,
        ],
    )
    def sc_embed(idx_hbm, tab_hbm, out_hbm, tab_v, bld_v, t012_sh, idx_v,
                 cidx_v, out_v0, out_v1, in_s0, in_s1, g_s0, g_s1, o_s0, o_s1):
        cid = lax.axis_index("c")
        sid = lax.axis_index("s")
        wid = sid * NC + cid
        base = wid * RW
        out_v = (out_v0, out_v1)
        in_s = (in_s0, in_s1)
        g_s = (g_s0, g_s1)
        o_s = (o_s0, o_s1)

        pltpu.sync_copy(tab_hbm, tab_v)

        r0 = sid * BLD
        a0 = r0 // (MAXV * MAXV)
        rem = r0 - a0 * (MAXV * MAXV)
        b0 = rem // MAXV
        c0 = rem - b0 * MAXV

        def build(j, abc):
            a, b, c = abc
            for k in range(NVEC):
                s = pl.ds(k * NLANE, NLANE)
                bld_v[j, s] = tab_v[a, s] + tab_v[MAXV + b, s] + tab_v[2 * MAXV + c, s]
            roll_c = c == MAXV - 1
            roll_b = jnp.logical_and(roll_c, b == MAXV - 1)
            c = jnp.where(roll_c, 0, c + 1)
            b = jnp.where(roll_c, jnp.where(roll_b, 0, b + 1), b)
            a = jnp.where(roll_b, a + 1, a)
            return (a, b, c)

        lax.fori_loop(0, BLD, build, (a0, b0, c0))
        pltpu.sync_copy(bld_v, t012_sh.at[pl.ds(r0, BLD), :])
        plsc.subcore_barrier()

        def start_in(g, b):
            return pltpu.async_copy(
                idx_hbm.at[:, pl.ds(base + g * CH, CH)], idx_v.at[b], in_s[b]
            )

        start_in(0, 0)

        def step(t, _):
            for b in range(2):
                g = 2 * t + b
                pltpu.make_async_copy(
                    idx_hbm.at[:, pl.ds(base, CH)], idx_v.at[b], in_s[b]
                ).wait()

                @pl.when(g + 1 < NCH)
                def _():
                    start_in(g + 1, 1 - b)

                for gi in range(CH // NLANE):
                    q, off = divmod(gi * NLANE, GSZ)
                    iv0 = idx_v[b, 0, pl.ds(gi * NLANE, NLANE)]
                    iv1 = idx_v[b, 1, pl.ds(gi * NLANE, NLANE)]
                    iv2 = idx_v[b, 2, pl.ds(gi * NLANE, NLANE)]
                    cidx_v[b, q, pl.ds(off, NLANE)] = (
                        iv0 * MAXV + iv1
                    ) * MAXV + iv2

                @pl.when(g >= 2)
                def _():
                    pltpu.make_async_copy(
                        out_v[b], out_hbm.at[pl.ds(base, CH), :], o_s[b]
                    ).wait()

                for q in range(2):
                    pltpu.async_copy(
                        t012_sh.at[cidx_v.at[b, q]],
                        out_v[b].at[pl.ds(q * GSZ, GSZ), :],
                        g_s[b],
                    ).wait()
                pltpu.async_copy(
                    out_v[b], out_hbm.at[pl.ds(base + g * CH, CH), :], o_s[b]
                )
            return 0

        lax.fori_loop(0, NCH // 2, step, 0)
        for b in range(2):
            pltpu.make_async_copy(
                out_v[b], out_hbm.at[pl.ds(base, CH), :], o_s[b]
            ).wait()

    return sc_embed


def kernel(inputs, emb_weight):
    B, X, Y, C = inputs.shape
    R = B * X * Y
    idx = inputs.reshape(R, C).astype(jnp.int32).T
    out = _build_sc_call(R)(idx, emb_weight)
    return out.reshape(B, X, Y, D)


# speedup vs baseline: 15.0834x; 1.1962x over previous
import functools

import jax
import jax.numpy as jnp
from jax import lax
from jax.experimental import pallas as pl
from jax.experimental.pallas import tpu as pltpu
from jax.experimental.pallas import tpu_sc as plsc

MAXV = 11
D = 128
NLANE = 16
NVEC = D // NLANE
NCOMBO = MAXV * MAXV * MAXV
BLD = 84
SLAB = 7
CSLAB = 16
CH = SLAB * CSLAB


@functools.cache
def _build_sc_call(R: int):
    NC, NS = 2, 16
    NW = NC * NS
    RW = R // NW
    NCH = RW // CH
    SW = RW // SLAB
    assert RW % CH == 0 and NCH % 8 == 0

    mesh = plsc.VectorSubcoreMesh(
        core_axis_name="c", subcore_axis_name="s", num_cores=NC, num_subcores=NS
    )

    @functools.partial(
        pl.kernel,
        out_type=jax.ShapeDtypeStruct((R // SLAB, SLAB, D), jnp.float32),
        mesh=mesh,
        compiler_params=pltpu.CompilerParams(use_tc_tiling_on_sc=True),
        scratch_types=[
            pltpu.VMEM((3 * MAXV, D), jnp.float32),
            pltpu.VMEM((BLD, D), jnp.float32),
            pltpu.VMEM_SHARED((16 * BLD, D), jnp.float32),
            pltpu.VMEM((3, 8 * 8 * CSLAB), jnp.int32),
            pltpu.VMEM((8 * CSLAB,), jnp.int32),
            pltpu.VMEM((8 * CSLAB,), jnp.int32),
            pltpu.VMEM((CSLAB, SLAB, D), jnp.float32),
            pltpu.VMEM((CSLAB, SLAB, D), jnp.float32),
            pltpu.SemaphoreType.DMA,
            pltpu.SemaphoreType.DMA,
            pltpu.SemaphoreType.DMA,
        ],
    )
    def sc_embed(idx_hbm, tab_hbm, out_hbm, tab_v, bld_v, t012_sh,
                 idx_v, cidx_v0, cidx_v1, out_v0, out_v1,
                 g_s0, o_s0, o_s1):
        cid = lax.axis_index("c")
        sid = lax.axis_index("s")
        wid = sid * NC + cid
        base = wid * RW
        sbase0 = wid * SW
        cidx_v = (cidx_v0, cidx_v1)
        out_v = (out_v0, out_v1)
        o_s = (o_s0, o_s1)

        pltpu.sync_copy(tab_hbm, tab_v)

        r0 = sid * BLD
        a0 = r0 // (MAXV * MAXV)
        rem = r0 - a0 * (MAXV * MAXV)
        b0 = rem // MAXV
        c0 = rem - b0 * MAXV

        def build(j, abc):
            a, b, c = abc
            for k in range(NVEC):
                s = pl.ds(k * NLANE, NLANE)
                bld_v[j, s] = tab_v[a, s] + tab_v[MAXV + b, s] + tab_v[2 * MAXV + c, s]
            roll_c = c == MAXV - 1
            roll_b = jnp.logical_and(roll_c, b == MAXV - 1)
            c = jnp.where(roll_c, 0, c + 1)
            b = jnp.where(roll_c, jnp.where(roll_b, 0, b + 1), b)
            a = jnp.where(roll_b, a + 1, a)
            return (a, b, c)

        lax.fori_loop(0, BLD, build, (a0, b0, c0))
        pltpu.sync_copy(bld_v, t012_sh.at[pl.ds(r0, BLD), :])
        plsc.subcore_barrier()

        CP = 8 * CSLAB
        basep = wid * SW * 8

        def window(w, _):
            pltpu.sync_copy(idx_hbm.at[:, pl.ds(basep + w * 8 * CP, 8 * CP)], idx_v)
            for sc in range(8):
                b = sc % 2
                for p in range(CP // NLANE):
                    off = sc * CP + p * NLANE
                    i0 = idx_v[0, pl.ds(off, NLANE)]
                    i1 = idx_v[1, pl.ds(off, NLANE)]
                    i2 = idx_v[2, pl.ds(off, NLANE)]
                    cidx_v[b][pl.ds(p * NLANE, NLANE)] = (i0 * MAXV + i1) * MAXV + i2

                def drain_out():
                    pltpu.make_async_copy(
                        out_v[b], out_hbm.at[pl.ds(sbase0, CSLAB), :, :], o_s[b]
                    ).wait()

                if sc < 2:
                    @pl.when(w > 0)
                    def _():
                        drain_out()
                else:
                    drain_out()

                descs = [
                    pltpu.async_copy(
                        t012_sh.at[cidx_v[b].at[pl.ds(s * 8, SLAB)]],
                        out_v[b].at[s],
                        g_s0,
                    )
                    for s in range(CSLAB)
                ]
                for d in descs:
                    d.wait()
                pltpu.async_copy(
                    out_v[b],
                    out_hbm.at[pl.ds(sbase0 + (w * 8 + sc) * CSLAB, CSLAB), :, :],
                    o_s[b],
                )
            return 0

        lax.fori_loop(0, NCH // 8, window, 0)
        for b in range(2):
            pltpu.make_async_copy(
                out_v[b], out_hbm.at[pl.ds(sbase0, CSLAB), :, :], o_s[b]
            ).wait()

    return sc_embed


def kernel(inputs, emb_weight):
    B, X, Y, C = inputs.shape
    R = B * X * Y
    idx = inputs.reshape(B * X, Y, C).astype(jnp.int32)
    idx = jnp.pad(idx, ((0, 0), (0, 8 - Y), (0, 0)), mode="edge")
    idx = idx.reshape(B * X * 8, C).T
    out = _build_sc_call(R)(idx, emb_weight)
    return out.reshape(B, X, Y, D)


# speedup vs baseline: 55.3100x; 3.6669x over previous
import functools

import jax
import jax.numpy as jnp
from jax import lax
from jax.experimental import pallas as pl
from jax.experimental.pallas import tpu as pltpu
from jax.experimental.pallas import tpu_sc as plsc

MAXV = 11
D = 128
NLANE = 16
NVEC = D // NLANE
BLD = 84
CB = 256
GSZ = 128


@functools.cache
def _build_sc_call(B: int, XY: int):
    NC, NS = 2, 16
    NW = NC * NS
    BW = B // NW
    NSUB = BW // CB
    assert B % NW == 0 and BW % CB == 0 and NSUB == 2 and CB % GSZ == 0

    mesh = plsc.VectorSubcoreMesh(
        core_axis_name="c", subcore_axis_name="s", num_cores=NC, num_subcores=NS
    )

    @functools.partial(
        pl.kernel,
        out_type=jax.ShapeDtypeStruct((XY, B, D), jnp.float32),
        mesh=mesh,
        compiler_params=pltpu.CompilerParams(use_tc_tiling_on_sc=True),
        scratch_types=[
            pltpu.VMEM((3 * MAXV, D), jnp.float32),
            pltpu.VMEM((BLD, D), jnp.float32),
            pltpu.VMEM_SHARED((16 * BLD, D), jnp.float32),
            pltpu.VMEM((3, BW), jnp.int32),
            pltpu.VMEM((CB,), jnp.int32),
            pltpu.VMEM((CB,), jnp.int32),
            pltpu.VMEM((CB, D), jnp.float32),
            pltpu.VMEM((CB, D), jnp.float32),
            pltpu.SemaphoreType.DMA,
            pltpu.SemaphoreType.DMA,
            pltpu.SemaphoreType.DMA,
        ],
    )
    def sc_embed(idx_hbm, tab_hbm, out_hbm, tab_v, bld_v, t012_sh,
                 idx_v, cidx_v0, cidx_v1, out_v0, out_v1, g_s0, o_s0, o_s1):
        cid = lax.axis_index("c")
        sid = lax.axis_index("s")
        wid = sid * NC + cid
        b0w = wid * BW
        cidx_v = (cidx_v0, cidx_v1)
        out_v = (out_v0, out_v1)
        o_s = (o_s0, o_s1)

        pltpu.sync_copy(tab_hbm, tab_v)

        r0 = sid * BLD
        a0 = r0 // (MAXV * MAXV)
        rem = r0 - a0 * (MAXV * MAXV)
        b0 = rem // MAXV
        c0 = rem - b0 * MAXV

        def build(j, abc):
            a, b, c = abc
            for k in range(NVEC):
                s = pl.ds(k * NLANE, NLANE)
                bld_v[j, s] = tab_v[a, s] + tab_v[MAXV + b, s] + tab_v[2 * MAXV + c, s]
            roll_c = c == MAXV - 1
            roll_b = jnp.logical_and(roll_c, b == MAXV - 1)
            c = jnp.where(roll_c, 0, c + 1)
            b = jnp.where(roll_c, jnp.where(roll_b, 0, b + 1), b)
            a = jnp.where(roll_b, a + 1, a)
            return (a, b, c)

        lax.fori_loop(0, BLD, build, (a0, b0, c0))
        pltpu.sync_copy(bld_v, t012_sh.at[pl.ds(r0, BLD), :])
        plsc.subcore_barrier()

        def window(xy, _):
            pltpu.sync_copy(idx_hbm.at[:, xy, pl.ds(b0w, BW)], idx_v)
            for sub in range(NSUB):
                b = sub
                for p in range(CB // NLANE):
                    off = sub * CB + p * NLANE
                    i0 = idx_v[0, pl.ds(off, NLANE)]
                    i1 = idx_v[1, pl.ds(off, NLANE)]
                    i2 = idx_v[2, pl.ds(off, NLANE)]
                    cidx_v[b][pl.ds(p * NLANE, NLANE)] = (i0 * MAXV + i1) * MAXV + i2

                @pl.when(xy > 0)
                def _():
                    pltpu.make_async_copy(
                        out_v[b], out_hbm.at[0, pl.ds(b0w, CB), :], o_s[b]
                    ).wait()

                descs = [
                    pltpu.async_copy(
                        t012_sh.at[cidx_v[b].at[pl.ds(q * GSZ, GSZ)]],
                        out_v[b].at[pl.ds(q * GSZ, GSZ), :],
                        g_s0,
                    )
                    for q in range(CB // GSZ)
                ]
                for d in descs:
                    d.wait()
                pltpu.async_copy(
                    out_v[b],
                    out_hbm.at[xy, pl.ds(b0w + sub * CB, CB), :],
                    o_s[b],
                )
            return 0

        lax.fori_loop(0, XY, window, 0)
        for b in range(NSUB):
            pltpu.make_async_copy(
                out_v[b], out_hbm.at[0, pl.ds(b0w, CB), :], o_s[b]
            ).wait()

    return sc_embed


def kernel(inputs, emb_weight):
    B, X, Y, C = inputs.shape
    idx = inputs.reshape(B, X * Y, C).astype(jnp.int32).transpose(2, 1, 0)
    out = _build_sc_call(B, X * Y)(idx, emb_weight)
    return out.reshape(X, Y, B, D).transpose(2, 0, 1, 3)


# speedup vs baseline: 63.4007x; 1.1463x over previous
import functools

import jax
import jax.numpy as jnp
from jax import lax
from jax.experimental import pallas as pl
from jax.experimental.pallas import tpu as pltpu
from jax.experimental.pallas import tpu_sc as plsc

MAXV = 11
D = 128
NLANE = 16
NVEC = D // NLANE
BLD = 84
CB = 256
GSZ = 128


@functools.cache
def _build_sc_call(B: int, XY: int):
    NC, NS = 2, 16
    NW = NC * NS
    BW = B // NW
    NSUB = BW // CB
    assert B % NW == 0 and BW % CB == 0 and NSUB == 2 and CB % GSZ == 0

    mesh = plsc.VectorSubcoreMesh(
        core_axis_name="c", subcore_axis_name="s", num_cores=NC, num_subcores=NS
    )

    @functools.partial(
        pl.kernel,
        out_type=jax.ShapeDtypeStruct((XY, B, D), jnp.float32),
        mesh=mesh,
        compiler_params=pltpu.CompilerParams(use_tc_tiling_on_sc=True),
        scratch_types=[
            pltpu.VMEM((3 * MAXV, D), jnp.float32),
            pltpu.VMEM((BLD, D), jnp.float32),
            pltpu.VMEM_SHARED((16 * BLD, D), jnp.float32),
            pltpu.VMEM((3, BW), jnp.int32),
            pltpu.VMEM((3, BW), jnp.int32),
            pltpu.VMEM((CB,), jnp.int32),
            pltpu.VMEM((CB,), jnp.int32),
            pltpu.VMEM((CB, D), jnp.float32),
            pltpu.VMEM((CB, D), jnp.float32),
            pltpu.SemaphoreType.DMA,
            pltpu.SemaphoreType.DMA,
            pltpu.SemaphoreType.DMA,
            pltpu.SemaphoreType.DMA,
            pltpu.SemaphoreType.DMA,
        ],
    )
    def sc_embed(idx_hbm, tab_hbm, out_hbm, tab_v, bld_v, t012_sh,
                 idx_v0, idx_v1, cidx_v0, cidx_v1, out_v0, out_v1,
                 g_s0, o_s0, o_s1, i_s0, i_s1):
        cid = lax.axis_index("c")
        sid = lax.axis_index("s")
        wid = sid * NC + cid
        b0w = wid * BW
        idx_v = (idx_v0, idx_v1)
        cidx_v = (cidx_v0, cidx_v1)
        out_v = (out_v0, out_v1)
        o_s = (o_s0, o_s1)
        i_s = (i_s0, i_s1)

        pltpu.sync_copy(tab_hbm, tab_v)

        r0 = sid * BLD
        a0 = r0 // (MAXV * MAXV)
        rem = r0 - a0 * (MAXV * MAXV)
        b0 = rem // MAXV
        c0 = rem - b0 * MAXV

        def build(j, abc):
            a, b, c = abc
            for k in range(NVEC):
                s = pl.ds(k * NLANE, NLANE)
                bld_v[j, s] = tab_v[a, s] + tab_v[MAXV + b, s] + tab_v[2 * MAXV + c, s]
            roll_c = c == MAXV - 1
            roll_b = jnp.logical_and(roll_c, b == MAXV - 1)
            c = jnp.where(roll_c, 0, c + 1)
            b = jnp.where(roll_c, jnp.where(roll_b, 0, b + 1), b)
            a = jnp.where(roll_b, a + 1, a)
            return (a, b, c)

        lax.fori_loop(0, BLD, build, (a0, b0, c0))
        pltpu.sync_copy(bld_v, t012_sh.at[pl.ds(r0, BLD), :])
        plsc.subcore_barrier()

        def start_in(xy, ib):
            pltpu.async_copy(idx_hbm.at[:, xy, pl.ds(b0w, BW)], idx_v[ib], i_s[ib])

        def process(xy, ib):
            for sub in range(NSUB):
                b = sub
                for p in range(CB // NLANE):
                    off = sub * CB + p * NLANE
                    i0 = idx_v[ib][0, pl.ds(off, NLANE)]
                    i1 = idx_v[ib][1, pl.ds(off, NLANE)]
                    i2 = idx_v[ib][2, pl.ds(off, NLANE)]
                    cidx_v[b][pl.ds(p * NLANE, NLANE)] = (i0 * MAXV + i1) * MAXV + i2

                @pl.when(xy > 0)
                def _():
                    pltpu.make_async_copy(
                        out_v[b], out_hbm.at[0, pl.ds(b0w, CB), :], o_s[b]
                    ).wait()

                descs = [
                    pltpu.async_copy(
                        t012_sh.at[cidx_v[b].at[pl.ds(q * GSZ, GSZ)]],
                        out_v[b].at[pl.ds(q * GSZ, GSZ), :],
                        g_s0,
                    )
                    for q in range(CB // GSZ)
                ]
                for d in descs:
                    d.wait()
                pltpu.async_copy(
                    out_v[b],
                    out_hbm.at[xy, pl.ds(b0w + sub * CB, CB), :],
                    o_s[b],
                )

        def wait_in(ib):
            pltpu.make_async_copy(
                idx_hbm.at[:, 0, pl.ds(b0w, BW)], idx_v[ib], i_s[ib]
            ).wait()

        start_in(0, 0)

        def pair(t, _):
            w0 = 2 * t
            wait_in(0)

            @pl.when(w0 + 1 < XY)
            def _():
                start_in(w0 + 1, 1)

            process(w0, 0)

            @pl.when(w0 + 1 < XY)
            def _():
                wait_in(1)

                @pl.when(w0 + 2 < XY)
                def _():
                    start_in(w0 + 2, 0)

                process(w0 + 1, 1)

            return 0

        lax.fori_loop(0, (XY + 1) // 2, pair, 0)
        for b in range(NSUB):
            pltpu.make_async_copy(
                out_v[b], out_hbm.at[0, pl.ds(b0w, CB), :], o_s[b]
            ).wait()

    return sc_embed


def kernel(inputs, emb_weight):
    B, X, Y, C = inputs.shape
    idx = inputs.reshape(B, X * Y, C).astype(jnp.int32).transpose(2, 1, 0)
    out = _build_sc_call(B, X * Y)(idx, emb_weight)
    return out.reshape(X, Y, B, D).transpose(2, 0, 1, 3)
